# Initial kernel scaffold; baseline (speedup 1.0000x reference)
import functools

import jax
import jax.numpy as jnp
from jax import lax
from jax.experimental import pallas as pl
from jax.experimental.pallas import tpu as pltpu
from jax.experimental.pallas import tpu_sc as plsc

_N, _D, _E = 10000, 128, 320000
_NTILES = 32
_EPT = _E // _NTILES
_C = 80
_NCHUNK = _EPT // _C
_RPT = _N // 16
_BLK = 2000


def _prep_body(x_ref, v_ref, c_ref, s1_ref, s2_ref, half_ref):
    xb = x_ref[...]
    s = jnp.dot(xb, v_ref[...], preferred_element_type=jnp.float32)
    s1 = s[:, 0:1] + c_ref[0, 0]
    s2 = s[:, 1:2] + c_ref[0, 1]
    z = s1 + s2
    att = jnp.maximum(z, 0.2 * z)
    s1_ref[...] = s1
    s2_ref[...] = s2
    half_ref[...] = (0.5 * att) * xb


_prep = pl.pallas_call(
    _prep_body,
    grid=(_N // _BLK,),
    in_specs=[
        pl.BlockSpec((_BLK, _D), lambda i: (i, 0)),
        pl.BlockSpec((_D, 2), lambda i: (0, 0)),
        pl.BlockSpec((1, 2), lambda i: (0, 0)),
    ],
    out_specs=[
        pl.BlockSpec((_BLK, 1), lambda i: (i, 0)),
        pl.BlockSpec((_BLK, 1), lambda i: (i, 0)),
        pl.BlockSpec((_BLK, _D), lambda i: (i, 0)),
    ],
    out_shape=[
        jax.ShapeDtypeStruct((_N, 1), jnp.float32),
        jax.ShapeDtypeStruct((_N, 1), jnp.float32),
        jax.ShapeDtypeStruct((_N, _D), jnp.float32),
    ],
)


_mesh = plsc.VectorSubcoreMesh(core_axis_name="c", subcore_axis_name="s")


@functools.partial(
    pl.kernel,
    mesh=_mesh,
    out_type=jax.ShapeDtypeStruct((2, _N, _D), jnp.float32),
    scratch_types=[
        pltpu.VMEM((_N,), jnp.float32),
        pltpu.VMEM((_N,), jnp.float32),
        pltpu.VMEM((_NCHUNK, _C), jnp.int32),
        pltpu.VMEM((_NCHUNK, _C), jnp.int32),
        pltpu.VMEM((_C,), jnp.float32),
        pltpu.VMEM((_C, _D), jnp.float32),
        pltpu.VMEM_SHARED((_N, _D), jnp.float32),
        pltpu.SemaphoreType.DMA,
    ],
)
def _edge_kernel(x_hbm, edges_hbm, s1_hbm, s2_hbm, half_hbm, out_hbm,
                 s1_v, s2_v, src_v, obj_v, att_v, rows_v, acc, sem):
    cid = lax.axis_index("c")
    sid = lax.axis_index("s")
    wid = sid * 2 + cid
    pltpu.sync_copy(s1_hbm, s1_v)
    pltpu.sync_copy(s2_hbm, s2_v)
    pltpu.sync_copy(edges_hbm.at[0, wid], src_v)
    pltpu.sync_copy(edges_hbm.at[1, wid], obj_v)
    pltpu.sync_copy(half_hbm.at[pl.ds(sid * _RPT, _RPT)],
                    acc.at[pl.ds(sid * _RPT, _RPT)])
    plsc.subcore_barrier()

    def chunk_body(j, carry):
        cp = pltpu.async_copy(x_hbm.at[obj_v.at[j]], rows_v, sem)
        for k in range(_C // 16):
            s_idx = src_v[j, pl.ds(k * 16, 16)]
            o_idx = obj_v[j, pl.ds(k * 16, 16)]
            z = plsc.load_gather(s1_v, [s_idx]) + plsc.load_gather(s2_v, [o_idx])
            att_v[pl.ds(k * 16, 16)] = jnp.maximum(z, 0.2 * z)
        cp.wait()

        def row_body(i, c2):
            a = att_v[i]
            for q in range(_D // 16):
                rows_v[i, pl.ds(q * 16, 16)] = a * rows_v[i, pl.ds(q * 16, 16)]
            return c2

        lax.fori_loop(0, _C, row_body, 0)
        pltpu.sync_copy(rows_v, acc.at[src_v.at[j]], add=True)
        return carry

    lax.fori_loop(0, _NCHUNK, chunk_body, 0)
    plsc.subcore_barrier()
    pltpu.sync_copy(acc.at[pl.ds(sid * _RPT, _RPT)],
                    out_hbm.at[cid, pl.ds(sid * _RPT, _RPT)])


def _combine_body(p_ref, o_ref):
    o_ref[...] = p_ref[0] + p_ref[1]


_combine = pl.pallas_call(
    _combine_body,
    grid=(_N // _BLK,),
    in_specs=[pl.BlockSpec((2, _BLK, _D), lambda i: (0, i, 0))],
    out_specs=pl.BlockSpec((_BLK, _D), lambda i: (i, 0)),
    out_shape=jax.ShapeDtypeStruct((_N, _D), jnp.float32),
)


def kernel(x, edge_index, W, b_w, a_w, a_b):
    n, d = x.shape
    a1 = a_w[0, :d]
    a2 = a_w[0, d:]
    v = jnp.stack([W.T @ a1, W.T @ a2], axis=1)
    c = jnp.stack([b_w @ a1 + a_b[0], b_w @ a2]).reshape(1, 2)
    s1_2d, s2_2d, half = _prep(x, v, c)
    s1 = s1_2d.reshape(n)
    s2 = s2_2d.reshape(n)
    edges_r = edge_index.reshape(2, _NTILES, _NCHUNK, _C)
    partial = _edge_kernel(x, edges_r, s1, s2, half)
    return _combine(partial)


# speedup vs baseline: 8.0556x; 8.0556x over previous
import functools

import jax
import jax.numpy as jnp
from jax import lax
from jax.experimental import pallas as pl
from jax.experimental.pallas import tpu as pltpu
from jax.experimental.pallas import tpu_sc as plsc

_N, _D, _E = 10000, 128, 320000
_NTILES = 32
_EPT = _E // _NTILES
_C = 80
_NCHUNK = _EPT // _C
_RPT = 624
_REM = _N - 16 * _RPT
_BLK = 2000


def _prep_body(x_ref, v_ref, c_ref, s1_ref, s2_ref, half_ref):
    xb = x_ref[...]
    s = jnp.dot(xb, v_ref[...], preferred_element_type=jnp.float32)
    s1 = s[:, 0:1] + c_ref[0, 0]
    s2 = s[:, 1:2] + c_ref[0, 1]
    z = s1 + s2
    att = jnp.maximum(z, 0.2 * z)
    s1_ref[...] = s1
    s2_ref[...] = s2
    half_ref[...] = (0.5 * att) * xb


_prep = pl.pallas_call(
    _prep_body,
    grid=(_N // _BLK,),
    in_specs=[
        pl.BlockSpec((_BLK, _D), lambda i: (i, 0)),
        pl.BlockSpec((_D, 2), lambda i: (0, 0)),
        pl.BlockSpec((1, 2), lambda i: (0, 0)),
    ],
    out_specs=[
        pl.BlockSpec((_BLK, 1), lambda i: (i, 0)),
        pl.BlockSpec((_BLK, 1), lambda i: (i, 0)),
        pl.BlockSpec((_BLK, _D), lambda i: (i, 0)),
    ],
    out_shape=[
        jax.ShapeDtypeStruct((_N, 1), jnp.float32),
        jax.ShapeDtypeStruct((_N, 1), jnp.float32),
        jax.ShapeDtypeStruct((_N, _D), jnp.float32),
    ],
)


_mesh = plsc.VectorSubcoreMesh(core_axis_name="c", subcore_axis_name="s")


@functools.partial(
    pl.kernel,
    mesh=_mesh,
    out_type=jax.ShapeDtypeStruct((2, _N, _D), jnp.float32),
    compiler_params=pltpu.CompilerParams(needs_layout_passes=False),
    scratch_types=[
        pltpu.VMEM((_NCHUNK, _C), jnp.int32),
        pltpu.VMEM((_NCHUNK, _C), jnp.int32),
        pltpu.VMEM((_C,), jnp.float32),
        pltpu.VMEM((_C,), jnp.float32),
        pltpu.VMEM((_C + 16,), jnp.float32),
        pltpu.VMEM((_C, _D), jnp.float32),
        pltpu.VMEM_SHARED((_N, _D), jnp.float32),
        pltpu.SemaphoreType.DMA,
    ],
)
def _edge_kernel(x_hbm, edges_hbm, s1_hbm, s2_hbm, half_hbm, out_hbm,
                 src_v, obj_v, g1_v, g2_v, att_v, rows_v, acc, sem):
    cid = lax.axis_index("c")
    sid = lax.axis_index("s")
    wid = sid * 2 + cid
    pltpu.sync_copy(edges_hbm.at[0, wid], src_v)
    pltpu.sync_copy(edges_hbm.at[1, wid], obj_v)
    pltpu.sync_copy(half_hbm.at[pl.ds(sid * _RPT, _RPT)],
                    acc.at[pl.ds(sid * _RPT, _RPT)])

    @pl.when(sid == 0)
    def _():
        pltpu.sync_copy(half_hbm.at[pl.ds(16 * _RPT, _REM)],
                        acc.at[pl.ds(16 * _RPT, _REM)])

    plsc.subcore_barrier()

    def chunk_body(j, carry):
        cp1 = pltpu.async_copy(s1_hbm.at[src_v.at[j]], g1_v, sem)
        cp2 = pltpu.async_copy(s2_hbm.at[obj_v.at[j]], g2_v, sem)
        cp3 = pltpu.async_copy(x_hbm.at[obj_v.at[j]], rows_v, sem)
        cp1.wait()
        cp2.wait()
        for k in range(_C // 16):
            z = g1_v[pl.ds(k * 16, 16)] + g2_v[pl.ds(k * 16, 16)]
            att_v[pl.ds(k * 16, 16)] = jnp.maximum(z, 0.2 * z)
        cp3.wait()

        def row_body(i, c2):
            a = att_v[pl.ds(i, 16)][0]
            for q in range(_D // 16):
                rows_v[i, pl.ds(q * 16, 16)] = a * rows_v[i, pl.ds(q * 16, 16)]
            return c2

        lax.fori_loop(0, _C, row_body, 0)
        pltpu.sync_copy(rows_v, acc.at[src_v.at[j]], add=True)
        return carry

    lax.fori_loop(0, _NCHUNK, chunk_body, 0)
    plsc.subcore_barrier()
    pltpu.sync_copy(acc.at[pl.ds(sid * _RPT, _RPT)],
                    out_hbm.at[cid, pl.ds(sid * _RPT, _RPT)])

    @pl.when(sid == 0)
    def _():
        pltpu.sync_copy(acc.at[pl.ds(16 * _RPT, _REM)],
                        out_hbm.at[cid, pl.ds(16 * _RPT, _REM)])


def _combine_body(p_ref, o_ref):
    o_ref[...] = p_ref[0] + p_ref[1]


_combine = pl.pallas_call(
    _combine_body,
    grid=(_N // _BLK,),
    in_specs=[pl.BlockSpec((2, _BLK, _D), lambda i: (0, i, 0))],
    out_specs=pl.BlockSpec((_BLK, _D), lambda i: (i, 0)),
    out_shape=jax.ShapeDtypeStruct((_N, _D), jnp.float32),
)


def kernel(x, edge_index, W, b_w, a_w, a_b):
    n, d = x.shape
    a1 = a_w[0, :d]
    a2 = a_w[0, d:]
    v = jnp.stack([W.T @ a1, W.T @ a2], axis=1)
    c = jnp.stack([b_w @ a1 + a_b[0], b_w @ a2]).reshape(1, 2)
    s1_2d, s2_2d, half = _prep(x, v, c)
    s1 = s1_2d.reshape(n)
    s2 = s2_2d.reshape(n)
    edges_r = edge_index.reshape(2, _NTILES, _NCHUNK, _C)
    partial = _edge_kernel(x, edges_r, s1, s2, half)
    return _combine(partial)
